# baseline (speedup 1.0000x reference)
import jax
import jax.numpy as jnp
from jax.experimental import pallas as pl
from jax.experimental.pallas import tpu as pltpu

_B, _N, _NFEAT, _H, _NLAYERS = 8, 256, 32, 32, 2
_G = 8
_RT = _N // _G


def _silu(v):
    a = 0.5 * v
    return a * jnp.tanh(a) + a


def _mm3(w, v):
    f, r, c = v.shape
    return jnp.dot(w, v.reshape(f, r * c).astype(jnp.bfloat16),
                   preferred_element_type=jnp.float32).reshape(f, r, c)


def _egnn_kernel(nfT_ref, adj_ref, vr_ref, vc_ref, xT_ref, WembT_ref,
                 *rest):
    layer_refs = rest[:-5]
    Wp1aT_ref, Wp1bT_ref, Wp2aT_ref, wp2b_ref, out_ref = rest[-5:]

    vr = vr_ref[0]
    vc = vc_ref[0]
    mask = adj_ref[0].astype(jnp.float32) * vr * vc

    hT = jnp.dot(WembT_ref[...], nfT_ref[0])
    xT = xT_ref[0]

    for l in range(_NLAYERS):
        (We1aT, We1bT, wd, We2blk, Wx1blk, wx2t,
         Wh1aT, Wh1bT, Wh2T) = (r[...] for r in layer_refs[9 * l:9 * l + 9])
        wd3 = wd[:, :, None]

        AT = jnp.dot(We1aT, hT)
        BT = jnp.dot(We1bT, hT)

        diff_tiles = []
        e1_tiles = []
        for g in range(_G):
            sl = slice(g * _RT, (g + 1) * _RT)
            Xi = xT[:, sl]
            diffT = Xi[:, :, None] - xT[:, None, :]
            dist2 = jnp.sum(diffT * diffT, axis=0)
            diff_tiles.append(diffT)
            e1_tiles.append(AT[:, sl][:, :, None] + BT[:, None, :]
                            + wd3 * dist2[None, :, :])

        E = jnp.concatenate(e1_tiles, axis=0)
        M8 = _silu(_mm3(We2blk, _silu(E)))
        mask8 = jnp.concatenate(
            [jnp.broadcast_to(mask[g * _RT:(g + 1) * _RT][None],
                              (_H, _RT, _N)) for g in range(_G)], axis=0)
        M8 = M8 * mask8
        T8 = _silu(_mm3(Wx1blk, M8))
        C4 = (T8 * wx2t[:, :, None]).reshape(_G, _H, _RT, _N).sum(axis=1)
        agg4 = jnp.sum(M8, axis=2)

        x_tiles = []
        agg_cols = []
        for g in range(_G):
            sl = slice(g * _RT, (g + 1) * _RT)
            cw = C4[g] * mask[sl]
            xupd = jnp.sum(diff_tiles[g] * cw[None], axis=2)
            x_tiles.append(xT[:, sl] + xupd * (1.0 / (_N - 1)))
            agg_cols.append(agg4[g * _H:(g + 1) * _H, :])

        xT = jnp.concatenate(x_tiles, axis=1)
        aggT = jnp.concatenate(agg_cols, axis=1)

        hT = hT + jnp.dot(Wh2T, _silu(jnp.dot(Wh1aT, hT)
                                      + jnp.dot(Wh1bT, aggT)))
        hT = hT * vr

    pT = jnp.dot(Wp1bT_ref[...], _silu(jnp.dot(Wp1aT_ref[...], hT)))
    pooled = jnp.sum(pT, axis=1, keepdims=True)
    z = _silu(jnp.dot(Wp2aT_ref[...], pooled))
    out_ref[...] = jnp.sum(z * wp2b_ref[...], axis=0,
                           keepdims=True)[None]


def _blockdiag(w, g):
    h = w.shape[0]
    out = jnp.zeros((g, h, g, h), w.dtype)
    for i in range(g):
        out = out.at[i, :, i, :].set(w)
    return out.reshape(g * h, g * h)


def kernel(node_feat, extra_unused, adj, valid, pos, W_emb, layers,
           Wp1a, Wp1b, Wp2a, Wp2b):
    b, n, nfeat = node_feat.shape
    h = W_emb.shape[1]

    nfT = jnp.swapaxes(node_feat, 1, 2)
    validf = valid.astype(jnp.float32)
    vr = validf.reshape(b, 1, n)
    vc = validf.reshape(b, n, 1)
    posT = jnp.swapaxes(pos, 1, 2)
    xT = jnp.pad(posT, ((0, 0), (0, 8 - posT.shape[1]), (0, 0)))

    weight_list = [W_emb.T]
    for p in layers:
        weight_list += [
            p["We1"][:h].T,
            p["We1"][h:2 * h].T,
            p["We1"][2 * h:].T,
            _blockdiag(p["We2"].T, _G).astype(jnp.bfloat16),
            _blockdiag(p["Wx1"].T, _G).astype(jnp.bfloat16),
            jnp.tile(p["Wx2"], (_G, 1)),
            p["Wh1"][:h].T, p["Wh1"][h:].T, p["Wh2"].T,
        ]
    weight_list += [Wp1a.T, Wp1b.T, Wp2a.T, Wp2b]

    def w_spec(arr):
        return pl.BlockSpec(arr.shape, lambda i: (0,) * arr.ndim)

    in_specs = [
        pl.BlockSpec((1, nfeat, n), lambda i: (i, 0, 0)),
        pl.BlockSpec((1, n, n), lambda i: (i, 0, 0)),
        pl.BlockSpec((1, 1, n), lambda i: (i, 0, 0)),
        pl.BlockSpec((1, n, 1), lambda i: (i, 0, 0)),
        pl.BlockSpec((1, 8, n), lambda i: (i, 0, 0)),
    ] + [w_spec(a) for a in weight_list]

    out = pl.pallas_call(
        _egnn_kernel,
        grid=(b,),
        in_specs=in_specs,
        out_specs=pl.BlockSpec((1, 1, 1), lambda i: (i, 0, 0)),
        out_shape=jax.ShapeDtypeStruct((b, 1, 1), jnp.float32),
        compiler_params=pltpu.CompilerParams(
            dimension_semantics=("parallel",),
        ),
    )(nfT, adj, vr, vc, xT, *weight_list)
    return out.reshape(b, 1)


# speedup vs baseline: 1.1169x; 1.1169x over previous
import jax
import jax.numpy as jnp
from jax.experimental import pallas as pl
from jax.experimental.pallas import tpu as pltpu

_B, _N, _NFEAT, _H, _NLAYERS = 8, 256, 32, 32, 2
_G = 8
_RT = _N // _G


def _silu(v):
    a = 0.5 * v
    return a * jnp.tanh(a) + a


def _mm3(w, v):
    f, r, c = v.shape
    return jnp.dot(w, v.reshape(f, r * c).astype(jnp.bfloat16),
                   preferred_element_type=jnp.float32).reshape(f, r, c)


def _egnn_kernel(nfT_ref, adj_ref, vr_ref, vc_ref, xT_ref, sqT_ref,
                 vec_ref, blk_ref, out_ref):
    vr = vr_ref[0]
    vc = vc_ref[0]
    mask = adj_ref[0].astype(jnp.float32) * vr * vc

    hT = jnp.dot(sqT_ref[0], nfT_ref[0])
    xT = xT_ref[0]

    for l in range(_NLAYERS):
        We1aT = sqT_ref[1 + 5 * l]
        We1bT = sqT_ref[2 + 5 * l]
        Wh1aT = sqT_ref[3 + 5 * l]
        Wh1bT = sqT_ref[4 + 5 * l]
        Wh2T = sqT_ref[5 + 5 * l]
        wd = vec_ref[2 * l]
        wx2t = vec_ref[2 * l + 1]
        We2blk = blk_ref[2 * l]
        Wx1blk = blk_ref[2 * l + 1]
        wd3 = wd[:, :, None]

        AT = jnp.dot(We1aT, hT)
        BT = jnp.dot(We1bT, hT)

        diff_tiles = []
        e1_tiles = []
        for g in range(_G):
            sl = slice(g * _RT, (g + 1) * _RT)
            Xi = xT[:, sl]
            diffT = Xi[:, :, None] - xT[:, None, :]
            dist2 = jnp.sum(diffT * diffT, axis=0)
            diff_tiles.append(diffT)
            e1_tiles.append(AT[:, sl][:, :, None] + BT[:, None, :]
                            + wd3 * dist2[None, :, :])

        E = jnp.concatenate(e1_tiles, axis=0)
        M8 = _silu(_mm3(We2blk, _silu(E)))
        mask8 = jnp.concatenate(
            [jnp.broadcast_to(mask[g * _RT:(g + 1) * _RT][None],
                              (_H, _RT, _N)) for g in range(_G)], axis=0)
        M8 = M8 * mask8
        T8 = _silu(_mm3(Wx1blk, M8))
        C4 = (T8.reshape(_G, _H, _RT, _N)
              * wx2t[None, :, :, None]).sum(axis=1)
        agg4 = jnp.sum(M8, axis=2)

        x_tiles = []
        agg_cols = []
        for g in range(_G):
            sl = slice(g * _RT, (g + 1) * _RT)
            cw = C4[g] * mask[sl]
            xupd = jnp.sum(diff_tiles[g] * cw[None], axis=2)
            x_tiles.append(xT[:, sl] + xupd * (1.0 / (_N - 1)))
            agg_cols.append(agg4[g * _H:(g + 1) * _H, :])

        xT = jnp.concatenate(x_tiles, axis=1)
        aggT = jnp.concatenate(agg_cols, axis=1)

        hT = hT + jnp.dot(Wh2T, _silu(jnp.dot(Wh1aT, hT)
                                      + jnp.dot(Wh1bT, aggT)))
        hT = hT * vr

    pT = jnp.dot(sqT_ref[12], _silu(jnp.dot(sqT_ref[11], hT)))
    pooled = jnp.sum(pT, axis=1, keepdims=True)
    z = _silu(jnp.dot(sqT_ref[13], pooled))
    out_ref[...] = jnp.sum(z * vec_ref[4], axis=0,
                           keepdims=True)[None]


def kernel(node_feat, extra_unused, adj, valid, pos, W_emb, layers,
           Wp1a, Wp1b, Wp2a, Wp2b):
    b, n, nfeat = node_feat.shape
    h = W_emb.shape[1]

    nfT = jnp.swapaxes(node_feat, 1, 2)
    validf = valid.astype(jnp.float32)
    vr = validf.reshape(b, 1, n)
    vc = validf.reshape(b, n, 1)
    posT = jnp.swapaxes(pos, 1, 2)
    xT = jnp.pad(posT, ((0, 0), (0, 8 - posT.shape[1]), (0, 0)))

    sq = [W_emb]
    for p in layers:
        sq += [p["We1"][:h], p["We1"][h:2 * h],
               p["Wh1"][:h], p["Wh1"][h:], p["Wh2"]]
    sq += [Wp1a, Wp1b, Wp2a]
    sqT = jnp.transpose(jnp.stack(sq), (0, 2, 1))

    vec = jnp.stack([
        layers[0]["We1"][2 * h].reshape(h, 1),
        layers[0]["Wx2"],
        layers[1]["We1"][2 * h].reshape(h, 1),
        layers[1]["Wx2"],
        Wp2b,
    ])

    ws = jnp.transpose(jnp.stack([
        layers[0]["We2"], layers[0]["Wx1"],
        layers[1]["We2"], layers[1]["Wx1"],
    ]), (0, 2, 1))
    eye = jnp.eye(_G, dtype=jnp.float32)
    blk = (eye[None, :, None, :, None] * ws[:, None, :, None, :]
           ).reshape(4, _G * h, _G * h).astype(jnp.bfloat16)

    in_specs = [
        pl.BlockSpec((1, nfeat, n), lambda i: (i, 0, 0)),
        pl.BlockSpec((1, n, n), lambda i: (i, 0, 0)),
        pl.BlockSpec((1, 1, n), lambda i: (i, 0, 0)),
        pl.BlockSpec((1, n, 1), lambda i: (i, 0, 0)),
        pl.BlockSpec((1, 8, n), lambda i: (i, 0, 0)),
        pl.BlockSpec(sqT.shape, lambda i: (0, 0, 0)),
        pl.BlockSpec(vec.shape, lambda i: (0, 0, 0)),
        pl.BlockSpec(blk.shape, lambda i: (0, 0, 0)),
    ]

    out = pl.pallas_call(
        _egnn_kernel,
        grid=(b,),
        in_specs=in_specs,
        out_specs=pl.BlockSpec((1, 1, 1), lambda i: (i, 0, 0)),
        out_shape=jax.ShapeDtypeStruct((b, 1, 1), jnp.float32),
        compiler_params=pltpu.CompilerParams(
            dimension_semantics=("parallel",),
        ),
    )(nfT, adj, vr, vc, xT, sqT, vec, blk)
    return out.reshape(b, 1)


# speedup vs baseline: 1.2585x; 1.1268x over previous
import jax
import jax.numpy as jnp
from jax.experimental import pallas as pl
from jax.experimental.pallas import tpu as pltpu

_B, _N, _NFEAT, _H, _NLAYERS = 8, 256, 32, 32, 2
_G = 8
_RT = _N // _G


def _silu(v):
    a = 0.5 * v
    return a * jnp.tanh(a) + a


def _mm3(w, v):
    f, r, c = v.shape
    return jnp.dot(w, v.reshape(f, r * c),
                   preferred_element_type=jnp.float32).reshape(f, r, c)


def _egnn_kernel(nfT_ref, adj_ref, vr_ref, vc_ref, xT_ref, sqT_ref,
                 vec_ref, blk_ref, out_ref):
    vr = vr_ref[0]
    vc = vc_ref[0]
    mask = adj_ref[0].astype(jnp.float32) * vr * vc
    mask2d = jnp.repeat(mask.reshape(_G, _RT * _N), _H, axis=0)

    hT = jnp.dot(sqT_ref[0], nfT_ref[0])
    xT = xT_ref[0]

    for l in range(_NLAYERS):
        We1aT = sqT_ref[1 + 5 * l]
        We1bT = sqT_ref[2 + 5 * l]
        Wh1aT = sqT_ref[3 + 5 * l]
        Wh1bT = sqT_ref[4 + 5 * l]
        Wh2T = sqT_ref[5 + 5 * l]
        wd = vec_ref[2 * l]
        wx2t = vec_ref[2 * l + 1]
        We2blk = blk_ref[2 * l]
        Wx1blk = blk_ref[2 * l + 1]
        wd3 = wd[:, :, None]

        AT = jnp.dot(We1aT, hT)
        BT = jnp.dot(We1bT, hT)

        diff_tiles = []
        e1_tiles = []
        for g in range(_G):
            sl = slice(g * _RT, (g + 1) * _RT)
            Xi = xT[:, sl]
            diffT = Xi[:, :, None] - xT[:, None, :]
            dist2 = jnp.sum(diffT * diffT, axis=0)
            diff_tiles.append(diffT)
            e1_tiles.append(AT[:, sl][:, :, None] + BT[:, None, :]
                            + wd3 * dist2[None, :, :])

        E = jnp.concatenate(e1_tiles, axis=0)
        E2 = _silu(E).reshape(_G * _H, _RT * _N)
        M2 = _silu(jnp.dot(We2blk, E2,
                           preferred_element_type=jnp.float32)) * mask2d
        T2 = _silu(jnp.dot(Wx1blk, M2,
                           preferred_element_type=jnp.float32))
        C8 = (T2.reshape(_G, _H, _RT * _N)
              * wx2t[None, :, :]).sum(axis=1)
        C4 = C8.reshape(_G, _RT, _N)
        agg4 = jnp.sum(M2.reshape(_G * _H, _RT, _N), axis=2)

        x_tiles = []
        agg_cols = []
        for g in range(_G):
            sl = slice(g * _RT, (g + 1) * _RT)
            cw = C4[g] * mask[sl]
            xupd = jnp.sum(diff_tiles[g] * cw[None], axis=2)
            x_tiles.append(xT[:, sl] + xupd * (1.0 / (_N - 1)))
            agg_cols.append(agg4[g * _H:(g + 1) * _H, :])

        xT = jnp.concatenate(x_tiles, axis=1)
        aggT = jnp.concatenate(agg_cols, axis=1)

        hT = hT + jnp.dot(Wh2T, _silu(jnp.dot(Wh1aT, hT)
                                      + jnp.dot(Wh1bT, aggT)))
        hT = hT * vr

    pT = jnp.dot(sqT_ref[12], _silu(jnp.dot(sqT_ref[11], hT)))
    pooled = jnp.sum(pT, axis=1, keepdims=True)
    z = _silu(jnp.dot(sqT_ref[13], pooled))
    out_ref[...] = jnp.sum(z * vec_ref[4], axis=0,
                           keepdims=True)[None]


def kernel(node_feat, extra_unused, adj, valid, pos, W_emb, layers,
           Wp1a, Wp1b, Wp2a, Wp2b):
    b, n, nfeat = node_feat.shape
    h = W_emb.shape[1]

    nfT = jnp.swapaxes(node_feat, 1, 2)
    validf = valid.astype(jnp.float32)
    vr = validf.reshape(b, 1, n)
    vc = validf.reshape(b, n, 1)
    posT = jnp.swapaxes(pos, 1, 2)
    xT = jnp.pad(posT, ((0, 0), (0, 8 - posT.shape[1]), (0, 0)))

    sq = [W_emb]
    for p in layers:
        sq += [p["We1"][:h], p["We1"][h:2 * h],
               p["Wh1"][:h], p["Wh1"][h:], p["Wh2"]]
    sq += [Wp1a, Wp1b, Wp2a]
    sqT = jnp.transpose(jnp.stack(sq), (0, 2, 1))

    vec = jnp.stack([
        layers[0]["We1"][2 * h].reshape(h, 1),
        layers[0]["Wx2"],
        layers[1]["We1"][2 * h].reshape(h, 1),
        layers[1]["Wx2"],
        Wp2b,
    ])

    ws = jnp.transpose(jnp.stack([
        layers[0]["We2"], layers[0]["Wx1"],
        layers[1]["We2"], layers[1]["Wx1"],
    ]), (0, 2, 1))
    eye = jnp.eye(_G, dtype=jnp.float32)
    blk = (eye[None, :, None, :, None] * ws[:, None, :, None, :]
           ).reshape(4, _G * h, _G * h)

    in_specs = [
        pl.BlockSpec((1, nfeat, n), lambda i: (i, 0, 0)),
        pl.BlockSpec((1, n, n), lambda i: (i, 0, 0)),
        pl.BlockSpec((1, 1, n), lambda i: (i, 0, 0)),
        pl.BlockSpec((1, n, 1), lambda i: (i, 0, 0)),
        pl.BlockSpec((1, 8, n), lambda i: (i, 0, 0)),
        pl.BlockSpec(sqT.shape, lambda i: (0, 0, 0)),
        pl.BlockSpec(vec.shape, lambda i: (0, 0, 0)),
        pl.BlockSpec(blk.shape, lambda i: (0, 0, 0)),
    ]

    out = pl.pallas_call(
        _egnn_kernel,
        grid=(b,),
        in_specs=in_specs,
        out_specs=pl.BlockSpec((1, 1, 1), lambda i: (i, 0, 0)),
        out_shape=jax.ShapeDtypeStruct((b, 1, 1), jnp.float32),
        compiler_params=pltpu.CompilerParams(
            dimension_semantics=("parallel",),
        ),
    )(nfT, adj, vr, vc, xT, sqT, vec, blk)
    return out.reshape(b, 1)
